# Initial kernel scaffold; baseline (speedup 1.0000x reference)
import functools

import numpy as np
import jax
import jax.numpy as jnp
from jax import lax
from jax.experimental import pallas as pl
from jax.experimental.pallas import tpu as pltpu
from jax.experimental.pallas import tpu_sc as plsc

N, H, W = 64, 512, 512
NUM_POINTS = 112 * 112
S = int(NUM_POINTS * 3.0)
K = int(0.75 * NUM_POINTS)
R = NUM_POINTS - K
P = S + R
PPAD = 40960
ROWS = PPAD // 128
AROWS = S // 128
NTILES = 32
CHUNKS = (2 * 4 * PPAD) // (128 * 128)
f32 = np.float32


def _build_tables():
    key = jax.random.key(42)
    k1, k2 = jax.random.split(key)
    pc = np.asarray(jax.random.uniform(k1, (N, S, 2), dtype=jnp.float32))
    rnd = np.asarray(jax.random.uniform(k2, (N, R, 2), dtype=jnp.float32))
    coords = np.concatenate([pc, rnd], axis=1)
    x = coords[..., 0] * f32(W) - f32(0.5)
    y = coords[..., 1] * f32(H) - f32(0.5)
    x0 = np.floor(x).astype(f32)
    y0 = np.floor(y).astype(f32)
    pix = np.clip(y0, 0, H - 1).astype(np.int64) * W + np.clip(x0, 0, W - 1).astype(np.int64)
    ordA = np.argsort(pix[:, :S], axis=1, kind="stable")
    ordB = np.argsort(pix[:, S:], axis=1, kind="stable") + S
    order = np.concatenate([ordA, ordB], axis=1)
    tk = np.take_along_axis
    x, y, x0, y0 = (tk(a, order, axis=1) for a in (x, y, x0, y0))
    wx1 = (x - x0).astype(f32); wx0 = (f32(1.0) - wx1).astype(f32)
    wy1 = (y - y0).astype(f32); wy0 = (f32(1.0) - wy1).astype(f32)
    idxs, ws = [], []
    for dy, wy in ((0, wy0), (1, wy1)):
        for dx, wx in ((0, wx0), (1, wx1)):
            ix = x0 + f32(dx); iy = y0 + f32(dy)
            valid = ((ix >= 0) & (ix <= W - 1) & (iy >= 0) & (iy <= H - 1))
            ixc = np.clip(ix, 0, W - 1).astype(np.int64)
            iyc = np.clip(iy, 0, H - 1).astype(np.int64)
            flat = (np.arange(N, dtype=np.int64)[:, None] * (H * W)
                    + iyc * W + ixc).astype(np.int32)
            idxs.append(flat)
            ws.append((wx * wy).astype(f32) * valid.astype(f32))
    IDX = np.stack(idxs, axis=1)
    WT = np.stack(ws, axis=1)
    padI = np.repeat(np.arange(N, dtype=np.int32)[:, None, None] * (H * W),
                     4, axis=1)
    IDXp = np.concatenate(
        [IDX, np.broadcast_to(padI, (N, 4, PPAD - P)).copy()], axis=2)
    WTp = np.concatenate([WT, np.zeros((N, 4, PPAD - P), f32)], axis=2)
    IDXp = IDXp.reshape(N, 4, ROWS, 128)
    WTp = WTp.reshape(N, 4, ROWS, 128)
    vb = np.zeros((ROWS - AROWS) * 128, f32)
    vb[: P - S] = 1.0
    VB = vb.reshape(ROWS - AROWS, 128)
    return (IDXp.reshape(NTILES, CHUNKS, 128, 128), WTp, VB)


_IDX_T, _WT, _VB = _build_tables()

_sc_mesh = plsc.VectorSubcoreMesh(core_axis_name="c", subcore_axis_name="s")


@functools.partial(
    pl.kernel,
    out_type=(jax.ShapeDtypeStruct((NTILES, CHUNKS, 128, 128), jnp.float32),
              jax.ShapeDtypeStruct((NTILES, CHUNKS, 128, 128), jnp.float32)),
    mesh=_sc_mesh,
    scratch_types=[
        pltpu.VMEM((128, 128), jnp.int32),
        pltpu.VMEM((128, 128), jnp.float32),
        pltpu.VMEM((128, 128), jnp.float32),
        pltpu.SemaphoreType.DMA,
        pltpu.SemaphoreType.DMA,
    ],
)
def _sc_gather(pf, tf, idx, outp, outt, idxv, valp, valt, semp, semt):
    wid = lax.axis_index("s") * 2 + lax.axis_index("c")
    for ch in range(CHUNKS):
        pltpu.sync_copy(idx.at[wid, ch], idxv)
        cp = pltpu.async_copy(pf.at[idxv], valp, semp)
        ct = pltpu.async_copy(tf.at[idxv], valt, semt)
        cp.wait()
        ct.wait()
        pltpu.sync_copy(valp, outp.at[wid, ch])
        pltpu.sync_copy(valt, outt.at[wid, ch])


def _tc_body(tp_ref, tt_ref, wt_ref, vb_ref, out_ref, acc):
    n = pl.program_id(0)

    @pl.when(n == 0)
    def _init():
        acc[0] = 0.0
        acc[1] = 0.0

    l = (tp_ref[0, 0] * wt_ref[0, 0] + tp_ref[0, 1] * wt_ref[0, 1]
         + tp_ref[0, 2] * wt_ref[0, 2] + tp_ref[0, 3] * wt_ref[0, 3])
    yv = (tt_ref[0, 0] * wt_ref[0, 0] + tt_ref[0, 1] * wt_ref[0, 1]
          + tt_ref[0, 2] * wt_ref[0, 2] + tt_ref[0, 3] * wt_ref[0, 3])
    lA = l[:AROWS]
    u = lax.bitcast_convert_type(jnp.abs(lA), jnp.int32)

    def bs_step(_, carry):
        lo, hi = carry
        mid = lo + (hi - lo) // 2
        c = jnp.sum(jnp.where(u <= mid, 1.0, 0.0))
        ge = c >= float(K)
        return (jnp.where(ge, lo, mid + 1), jnp.where(ge, mid, hi))

    lo, hi = lax.fori_loop(0, 31, bs_step,
                           (jnp.int32(0), jnp.int32(2**31 - 1)))
    t = lo
    c_lt = jnp.sum(jnp.where(u < t, 1.0, 0.0))
    c_eq = jnp.sum(jnp.where(u == t, 1.0, 0.0))
    w_eq = (float(K) - c_lt) / jnp.maximum(c_eq, 1.0)
    mA = jnp.where(u < t, 1.0, jnp.where(u == t, w_eq, 0.0))
    lB = l[AROWS:]
    yB = yv[AROWS:]
    yA = yv[:AROWS]
    mB = vb_ref[...]

    def terms(lv, yvv):
        bce = (jnp.maximum(lv, 0.0) - lv * yvv
               + jnp.log(1.0 + jnp.exp(-jnp.abs(lv))))
        pv = 1.0 / (1.0 + jnp.exp(-lv))
        return bce, pv

    bceA, pA = terms(lA, yA)
    bceB, pB = terms(lB, yB)
    s_bce = jnp.sum(mA * bceA) + jnp.sum(mB * bceB)
    s_py = jnp.sum(mA * pA * yA) + jnp.sum(mB * pB * yB)
    s_p = jnp.sum(mA * pA) + jnp.sum(mB * pB)
    s_y = jnp.sum(mA * yA) + jnp.sum(mB * yB)
    dice = 1.0 - (2.0 * s_py + 1.0) / (s_p + s_y + 1.0)
    acc[0] = acc[0] + s_bce
    acc[1] = acc[1] + dice

    @pl.when(n == N - 1)
    def _fin():
        loss_bce = acc[0] / float(N * NUM_POINTS)
        loss_dice = acc[1] / float(N)
        z = jnp.zeros((8, 128), jnp.float32)
        z = z.at[0, 0].set(loss_bce + loss_dice)
        z = z.at[0, 1].set(loss_bce)
        z = z.at[0, 2].set(loss_dice)
        out_ref[...] = z


_tc_reduce = pl.pallas_call(
    _tc_body,
    grid=(N,),
    in_specs=[
        pl.BlockSpec((1, 4, ROWS, 128), lambda n: (n, 0, 0, 0)),
        pl.BlockSpec((1, 4, ROWS, 128), lambda n: (n, 0, 0, 0)),
        pl.BlockSpec((1, 4, ROWS, 128), lambda n: (n, 0, 0, 0)),
        pl.BlockSpec((ROWS - AROWS, 128), lambda n: (0, 0)),
    ],
    out_specs=pl.BlockSpec((8, 128), lambda n: (0, 0)),
    out_shape=jax.ShapeDtypeStruct((8, 128), jnp.float32),
    scratch_shapes=[pltpu.SMEM((2,), jnp.float32)],
)


def kernel(pred, target):
    pf = pred.reshape(-1)
    tf = target.reshape(-1)
    tp, tt = _sc_gather(pf, tf, jnp.asarray(_IDX_T))
    tp = tp.reshape(N, 4, ROWS, 128)
    tt = tt.reshape(N, 4, ROWS, 128)
    out = _tc_reduce(tp, tt, jnp.asarray(_WT), jnp.asarray(_VB))
    return (out[0, 0], out[0, 1], out[0, 2])


# speedup vs baseline: 2.1722x; 2.1722x over previous
import functools

import numpy as np
import jax
import jax.numpy as jnp
from jax import lax
from jax.experimental import pallas as pl
from jax.experimental.pallas import tpu as pltpu
from jax.experimental.pallas import tpu_sc as plsc

N, H, W = 64, 512, 512
NUM_POINTS = 112 * 112
S = int(NUM_POINTS * 3.0)
K = int(0.75 * NUM_POINTS)
R = NUM_POINTS - K
P = S + R
PPAD = 40960
ROWS = PPAD // 128
AROWS = S // 128
NTILES = 32
CHUNKS = (2 * 4 * PPAD) // (128 * 128)
f32 = np.float32


def _threefry2x32(k1, k2, x0, x1):
    u32 = np.uint32

    def rol(x, d):
        return ((x << u32(d)) | (x >> u32(32 - d))).astype(u32)

    ks = (u32(k1), u32(k2), u32(k1) ^ u32(k2) ^ u32(0x1BD11BDA))
    x0 = (x0 + ks[0]).astype(u32)
    x1 = (x1 + ks[1]).astype(u32)
    r0, r1 = (13, 15, 26, 6), (17, 29, 16, 24)
    sched = ((r0, 1, 2, 1), (r1, 2, 0, 2), (r0, 0, 1, 3),
             (r1, 1, 2, 4), (r0, 2, 0, 5))
    for rots, ia, ib, inc in sched:
        for r in rots:
            x0 = (x0 + x1).astype(u32)
            x1 = x0 ^ rol(x1, r)
        x0 = (x0 + ks[ia]).astype(u32)
        x1 = (x1 + ks[ib] + u32(inc)).astype(u32)
    return x0, x1


def _np_random_bits(key, shape):
    size = int(np.prod(shape))
    io = np.arange(size, dtype=np.uint64)
    c1 = (io >> np.uint64(32)).astype(np.uint32)
    c2 = (io & np.uint64(0xFFFFFFFF)).astype(np.uint32)
    b1, b2 = _threefry2x32(key[0], key[1], c1, c2)
    return (b1 ^ b2).reshape(shape)


def _np_uniform(key, shape):
    bits = _np_random_bits(key, shape)
    fb = (bits >> np.uint32(9)) | np.uint32(0x3F800000)
    return (fb.view(f32) - f32(1.0)).astype(f32)


def _np_key42_split():
    key = np.array([0, 42], np.uint32)
    c1 = np.array([0, 0], np.uint32)
    c2 = np.array([0, 1], np.uint32)
    b1, b2 = _threefry2x32(key[0], key[1], c1, c2)
    return (np.array([b1[0], b2[0]], np.uint32),
            np.array([b1[1], b2[1]], np.uint32))


def _build_tables():
    k1, k2 = _np_key42_split()
    pc = _np_uniform(k1, (N, S, 2))
    rnd = _np_uniform(k2, (N, R, 2))
    coords = np.concatenate([pc, rnd], axis=1)
    x = coords[..., 0] * f32(W) - f32(0.5)
    y = coords[..., 1] * f32(H) - f32(0.5)
    x0 = np.floor(x).astype(f32)
    y0 = np.floor(y).astype(f32)
    pix = np.clip(y0, 0, H - 1).astype(np.int64) * W + np.clip(x0, 0, W - 1).astype(np.int64)
    ordA = np.argsort(pix[:, :S], axis=1, kind="stable")
    ordB = np.argsort(pix[:, S:], axis=1, kind="stable") + S
    order = np.concatenate([ordA, ordB], axis=1)
    tk = np.take_along_axis
    x, y, x0, y0 = (tk(a, order, axis=1) for a in (x, y, x0, y0))
    wx1 = (x - x0).astype(f32); wx0 = (f32(1.0) - wx1).astype(f32)
    wy1 = (y - y0).astype(f32); wy0 = (f32(1.0) - wy1).astype(f32)
    idxs, ws = [], []
    for dy, wy in ((0, wy0), (1, wy1)):
        for dx, wx in ((0, wx0), (1, wx1)):
            ix = x0 + f32(dx); iy = y0 + f32(dy)
            valid = ((ix >= 0) & (ix <= W - 1) & (iy >= 0) & (iy <= H - 1))
            ixc = np.clip(ix, 0, W - 1).astype(np.int64)
            iyc = np.clip(iy, 0, H - 1).astype(np.int64)
            flat = (np.arange(N, dtype=np.int64)[:, None] * (H * W)
                    + iyc * W + ixc).astype(np.int32)
            idxs.append(flat)
            ws.append((wx * wy).astype(f32) * valid.astype(f32))
    IDX = np.stack(idxs, axis=1)
    WT = np.stack(ws, axis=1)
    padI = np.repeat(np.arange(N, dtype=np.int32)[:, None, None] * (H * W),
                     4, axis=1)
    IDXp = np.concatenate(
        [IDX, np.broadcast_to(padI, (N, 4, PPAD - P)).copy()], axis=2)
    WTp = np.concatenate([WT, np.zeros((N, 4, PPAD - P), f32)], axis=2)
    IDXp = IDXp.reshape(N, 4, ROWS, 128)
    WTp = WTp.reshape(N, 4, ROWS, 128)
    vb = np.zeros((ROWS - AROWS) * 128, f32)
    vb[: P - S] = 1.0
    VB = vb.reshape(ROWS - AROWS, 128)
    return (IDXp.reshape(NTILES, CHUNKS, 128, 128), WTp, VB)


_IDX_T, _WT, _VB = _build_tables()

@functools.cache
def _get_sc_gather():
    mesh = plsc.VectorSubcoreMesh(core_axis_name="c", subcore_axis_name="s")

    @functools.partial(
        pl.kernel,
        out_type=(jax.ShapeDtypeStruct((NTILES, CHUNKS, 128, 128), jnp.float32),
                  jax.ShapeDtypeStruct((NTILES, CHUNKS, 128, 128), jnp.float32)),
        mesh=mesh,
        scratch_types=[
            pltpu.VMEM((128, 128), jnp.int32),
            pltpu.VMEM((128, 128), jnp.float32),
            pltpu.VMEM((128, 128), jnp.float32),
            pltpu.SemaphoreType.DMA,
            pltpu.SemaphoreType.DMA,
        ],
    )
    def _sc_gather(pf, tf, idx, outp, outt, idxv, valp, valt, semp, semt):
        wid = lax.axis_index("s") * 2 + lax.axis_index("c")

        def chunk(ch, carry):
            pltpu.sync_copy(idx.at[wid, ch], idxv)

            def body(g, c2):
                base = g * 8
                cps = []
                for i in range(8):
                    cps.append(pltpu.async_copy(
                        pf.at[idxv.at[base + i]], valp.at[base + i], semp))
                    cps.append(pltpu.async_copy(
                        tf.at[idxv.at[base + i]], valt.at[base + i], semt))
                for cp in cps:
                    cp.wait()
                return c2

            lax.fori_loop(0, 16, body, 0)
            pltpu.sync_copy(valp, outp.at[wid, ch])
            pltpu.sync_copy(valt, outt.at[wid, ch])
            return carry

        lax.fori_loop(0, CHUNKS, chunk, 0)

    return _sc_gather


def _tc_body(tp_ref, tt_ref, wt_ref, vb_ref, out_ref, acc):
    n = pl.program_id(0)

    @pl.when(n == 0)
    def _init():
        acc[0] = 0.0
        acc[1] = 0.0

    l = (tp_ref[0, 0] * wt_ref[0, 0] + tp_ref[0, 1] * wt_ref[0, 1]
         + tp_ref[0, 2] * wt_ref[0, 2] + tp_ref[0, 3] * wt_ref[0, 3])
    yv = (tt_ref[0, 0] * wt_ref[0, 0] + tt_ref[0, 1] * wt_ref[0, 1]
          + tt_ref[0, 2] * wt_ref[0, 2] + tt_ref[0, 3] * wt_ref[0, 3])
    lA = l[:AROWS]
    u = lax.bitcast_convert_type(jnp.abs(lA), jnp.int32)

    def bs_step(_, carry):
        lo, hi = carry
        mid = lo + (hi - lo) // 2
        c = jnp.sum(jnp.where(u <= mid, 1.0, 0.0))
        ge = c >= float(K)
        return (jnp.where(ge, lo, mid + 1), jnp.where(ge, mid, hi))

    lo, hi = lax.fori_loop(0, 31, bs_step,
                           (jnp.int32(0), jnp.int32(2**31 - 1)))
    t = lo
    c_lt = jnp.sum(jnp.where(u < t, 1.0, 0.0))
    c_eq = jnp.sum(jnp.where(u == t, 1.0, 0.0))
    w_eq = (float(K) - c_lt) / jnp.maximum(c_eq, 1.0)
    mA = jnp.where(u < t, 1.0, jnp.where(u == t, w_eq, 0.0))
    lB = l[AROWS:]
    yB = yv[AROWS:]
    yA = yv[:AROWS]
    mB = vb_ref[...]

    def terms(lv, yvv):
        bce = (jnp.maximum(lv, 0.0) - lv * yvv
               + jnp.log(1.0 + jnp.exp(-jnp.abs(lv))))
        pv = 1.0 / (1.0 + jnp.exp(-lv))
        return bce, pv

    bceA, pA = terms(lA, yA)
    bceB, pB = terms(lB, yB)
    s_bce = jnp.sum(mA * bceA) + jnp.sum(mB * bceB)
    s_py = jnp.sum(mA * pA * yA) + jnp.sum(mB * pB * yB)
    s_p = jnp.sum(mA * pA) + jnp.sum(mB * pB)
    s_y = jnp.sum(mA * yA) + jnp.sum(mB * yB)
    dice = 1.0 - (2.0 * s_py + 1.0) / (s_p + s_y + 1.0)
    acc[0] = acc[0] + s_bce
    acc[1] = acc[1] + dice

    @pl.when(n == N - 1)
    def _fin():
        loss_bce = acc[0] / float(N * NUM_POINTS)
        loss_dice = acc[1] / float(N)
        row = lax.broadcasted_iota(jnp.int32, (8, 128), 0)
        col = lax.broadcasted_iota(jnp.int32, (8, 128), 1)
        z = jnp.where(col == 0, loss_bce + loss_dice,
                      jnp.where(col == 1, loss_bce,
                                jnp.where(col == 2, loss_dice, 0.0)))
        out_ref[...] = jnp.where(row == 0, z, 0.0)


_tc_reduce = pl.pallas_call(
    _tc_body,
    grid=(N,),
    in_specs=[
        pl.BlockSpec((1, 4, ROWS, 128), lambda n: (n, 0, 0, 0)),
        pl.BlockSpec((1, 4, ROWS, 128), lambda n: (n, 0, 0, 0)),
        pl.BlockSpec((1, 4, ROWS, 128), lambda n: (n, 0, 0, 0)),
        pl.BlockSpec((ROWS - AROWS, 128), lambda n: (0, 0)),
    ],
    out_specs=pl.BlockSpec((8, 128), lambda n: (0, 0)),
    out_shape=jax.ShapeDtypeStruct((8, 128), jnp.float32),
    scratch_shapes=[pltpu.SMEM((2,), jnp.float32)],
)


def kernel(pred, target):
    pf = pred.reshape(-1)
    tf = target.reshape(-1)
    tp, tt = _get_sc_gather()(pf, tf, jnp.asarray(_IDX_T))
    tp = tp.reshape(N, 4, ROWS, 128)
    tt = tt.reshape(N, 4, ROWS, 128)
    out = _tc_reduce(tp, tt, jnp.asarray(_WT), jnp.asarray(_VB))
    return (out[0, 0], out[0, 1], out[0, 2])


# speedup vs baseline: 3.6107x; 1.6622x over previous
import functools

import numpy as np
import jax
import jax.numpy as jnp
from jax import lax
from jax.experimental import pallas as pl
from jax.experimental.pallas import tpu as pltpu
from jax.experimental.pallas import tpu_sc as plsc

N, H, W = 64, 512, 512
NUM_POINTS = 112 * 112
S = int(NUM_POINTS * 3.0)
K = int(0.75 * NUM_POINTS)
R = NUM_POINTS - K
P = S + R
NBANDS = 8
BROWS = H // NBANDS
BWORDS = (BROWS + 1) * W
f32 = np.float32


def _threefry2x32(k1, k2, x0, x1):
    u32 = np.uint32

    def rol(x, d):
        return ((x << u32(d)) | (x >> u32(32 - d))).astype(u32)

    ks = (u32(k1), u32(k2), u32(k1) ^ u32(k2) ^ u32(0x1BD11BDA))
    x0 = (x0 + ks[0]).astype(u32)
    x1 = (x1 + ks[1]).astype(u32)
    r0, r1 = (13, 15, 26, 6), (17, 29, 16, 24)
    sched = ((r0, 1, 2, 1), (r1, 2, 0, 2), (r0, 0, 1, 3),
             (r1, 1, 2, 4), (r0, 2, 0, 5))
    for rots, ia, ib, inc in sched:
        for r in rots:
            x0 = (x0 + x1).astype(u32)
            x1 = x0 ^ rol(x1, r)
        x0 = (x0 + ks[ia]).astype(u32)
        x1 = (x1 + ks[ib] + u32(inc)).astype(u32)
    return x0, x1


def _np_uniform(key, shape):
    size = int(np.prod(shape))
    io = np.arange(size, dtype=np.uint64)
    c1 = (io >> np.uint64(32)).astype(np.uint32)
    c2 = (io & np.uint64(0xFFFFFFFF)).astype(np.uint32)
    b1, b2 = _threefry2x32(key[0], key[1], c1, c2)
    bits = (b1 ^ b2).reshape(shape)
    fb = (bits >> np.uint32(9)) | np.uint32(0x3F800000)
    return (fb.view(f32) - f32(1.0)).astype(f32)


def _np_key42_split():
    key = np.array([0, 42], np.uint32)
    c1 = np.array([0, 0], np.uint32)
    c2 = np.array([0, 1], np.uint32)
    b1, b2 = _threefry2x32(key[0], key[1], c1, c2)
    return (np.array([b1[0], b2[0]], np.uint32),
            np.array([b1[1], b2[1]], np.uint32))


def _build_tables():
    k1, k2 = _np_key42_split()
    pc = _np_uniform(k1, (N, S, 2))
    rnd = _np_uniform(k2, (N, R, 2))
    coords = np.concatenate([pc, rnd], axis=1)
    x = coords[..., 0] * f32(W) - f32(0.5)
    y = coords[..., 1] * f32(H) - f32(0.5)
    x0 = np.floor(x).astype(f32)
    y0 = np.floor(y).astype(f32)
    x0c = np.clip(x0, 0, W - 1).astype(np.int64)
    y0c = np.clip(y0, 0, H - 1).astype(np.int64)
    pix = y0c * W + x0c
    ordA = np.argsort(pix[:, :S], axis=1, kind="stable")
    ordB = np.argsort(pix[:, S:], axis=1, kind="stable") + S
    order = np.concatenate([ordA, ordB], axis=1)
    tk = np.take_along_axis
    x, y, x0, y0 = (tk(a, order, axis=1) for a in (x, y, x0, y0))
    band = (np.clip(y0, 0, H - 1).astype(np.int32) // BROWS).astype(np.int32)
    wx1 = (x - x0).astype(f32); wx0 = (f32(1.0) - wx1).astype(f32)
    wy1 = (y - y0).astype(f32); wy0 = (f32(1.0) - wy1).astype(f32)
    lidx = np.empty((N, 4, P), np.int32)
    wts = np.empty((N, 4, P), f32)
    for t, (dy, dx, wy, wx) in enumerate(
            ((0, 0, wy0, wx0), (0, 1, wy0, wx1),
             (1, 0, wy1, wx0), (1, 1, wy1, wx1))):
        ix = x0 + f32(dx); iy = y0 + f32(dy)
        valid = ((ix >= 0) & (ix <= W - 1) & (iy >= 0) & (iy <= H - 1))
        ixc = np.clip(ix, 0, W - 1).astype(np.int32)
        iyc = np.clip(iy, 0, H - 1).astype(np.int32)
        lidx[:, t] = (iyc - BROWS * band) * W + ixc
        wts[:, t] = (wx * wy).astype(f32) * valid.astype(f32)
    cA = np.stack([(band[:, :S] == b).sum(axis=1) for b in range(NBANDS)])
    cB = np.stack([(band[:, S:] == b).sum(axis=1) for b in range(NBANDS)])
    capA = int(-(-int(cA.max()) // 128) * 128)
    capB = int(-(-int(cB.max()) // 128) * 128)
    cap = capA + capB
    IDX2 = np.zeros((N, NBANDS, 4 * cap), np.int32)
    WT2 = np.zeros((N, NBANDS, 4 * cap), f32)
    BIAS2 = np.zeros((N, NBANDS, cap), f32)
    BIAS2[..., :capA] = f32(1e30)
    BV = np.zeros((N, NBANDS, capB), f32)
    bnd = np.arange(NBANDS + 1)
    for n in range(N):
        eA = np.searchsorted(band[n, :S], bnd)
        eB = np.searchsorted(band[n, S:], bnd)
        i2 = IDX2[n].reshape(NBANDS, 4, cap)
        w2 = WT2[n].reshape(NBANDS, 4, cap)
        for b in range(NBANDS):
            ca = eA[b + 1] - eA[b]
            i2[b, :, :ca] = lidx[n, :, eA[b]:eA[b + 1]]
            w2[b, :, :ca] = wts[n, :, eA[b]:eA[b + 1]]
            BIAS2[n, b, :ca] = 0.0
            cb = eB[b + 1] - eB[b]
            i2[b, :, capA:capA + cb] = lidx[n, :, S + eB[b]:S + eB[b + 1]]
            w2[b, :, capA:capA + cb] = wts[n, :, S + eB[b]:S + eB[b + 1]]
            BV[n, b, :cb] = 1.0
    VB3 = BV.reshape(N, NBANDS * capB // 128, 128)
    return IDX2, WT2, BIAS2, VB3, capA, capB


_IDX2, _WT2, _BIAS2, _VB3, CAP_A, CAP_B = _build_tables()
CAP = CAP_A + CAP_B
ATOT = NBANDS * CAP_A
PTOT = NBANDS * CAP
AR2 = ATOT // 128
ROWS2 = PTOT // 128
BR2 = ROWS2 - AR2
assert (2 * BWORDS + 2 * 4 * CAP + 3 * CAP) * 4 <= 524284, "TileSpmem overflow"


@functools.cache
def _get_sc_sample():
    mesh = plsc.VectorSubcoreMesh(core_axis_name="c", subcore_axis_name="s")

    @functools.partial(
        pl.kernel,
        out_type=(jax.ShapeDtypeStruct((N, PTOT), jnp.float32),
                  jax.ShapeDtypeStruct((N, PTOT), jnp.float32)),
        mesh=mesh,
        compiler_params=pltpu.CompilerParams(needs_layout_passes=False),
        scratch_types=[
            pltpu.VMEM((BWORDS,), jnp.float32),
            pltpu.VMEM((BWORDS,), jnp.float32),
            pltpu.VMEM((4 * CAP,), jnp.int32),
            pltpu.VMEM((4 * CAP,), jnp.float32),
            pltpu.VMEM((CAP,), jnp.float32),
            pltpu.VMEM((CAP,), jnp.float32),
            pltpu.VMEM((CAP,), jnp.float32),
        ],
    )
    def _sc_sample(pr, tr, idx2, wt2, bias2, lout, yout,
                   bandp, bandt, idxb, wtb, biasb, lbuf, ybuf):
        wid = lax.axis_index("s") * 2 + lax.axis_index("c")
        for ii in range(2):
            img = wid * 2 + ii
            for b in range(NBANDS):
                rows = BROWS + 1 if b < NBANDS - 1 else BROWS
                nw = rows * W
                pltpu.sync_copy(pr.at[img, pl.ds(b * BROWS * W, nw)],
                                bandp.at[pl.ds(0, nw)])
                pltpu.sync_copy(tr.at[img, pl.ds(b * BROWS * W, nw)],
                                bandt.at[pl.ds(0, nw)])
                pltpu.sync_copy(idx2.at[img, b], idxb)
                pltpu.sync_copy(wt2.at[img, b], wtb)
                pltpu.sync_copy(bias2.at[img, b], biasb)

                def group(g, carry):
                    o = g * 16
                    accl = biasb[pl.ds(o, 16)]
                    accy = jnp.zeros((16,), jnp.float32)
                    for t in range(4):
                        iv = idxb[pl.ds(t * CAP + o, 16)]
                        wv = wtb[pl.ds(t * CAP + o, 16)]
                        pv = plsc.load_gather(bandp, [iv])
                        tv = plsc.load_gather(bandt, [iv])
                        accl = accl + wv * pv
                        accy = accy + wv * tv
                    lbuf[pl.ds(o, 16)] = accl
                    ybuf[pl.ds(o, 16)] = accy
                    return carry

                lax.fori_loop(0, CAP // 16, group, 0)
                pltpu.sync_copy(lbuf.at[pl.ds(0, CAP_A)],
                                lout.at[img, pl.ds(b * CAP_A, CAP_A)])
                pltpu.sync_copy(ybuf.at[pl.ds(0, CAP_A)],
                                yout.at[img, pl.ds(b * CAP_A, CAP_A)])
                pltpu.sync_copy(lbuf.at[pl.ds(CAP_A, CAP_B)],
                                lout.at[img, pl.ds(ATOT + b * CAP_B, CAP_B)])
                pltpu.sync_copy(ybuf.at[pl.ds(CAP_A, CAP_B)],
                                yout.at[img, pl.ds(ATOT + b * CAP_B, CAP_B)])

    return _sc_sample


def _tc_body(l_ref, y_ref, vb_ref, out_ref, acc):
    n = pl.program_id(0)

    @pl.when(n == 0)
    def _init():
        acc[0] = 0.0
        acc[1] = 0.0

    l = l_ref[0]
    yv = y_ref[0]
    lA = l[:AR2]
    u = lax.bitcast_convert_type(jnp.abs(lA), jnp.int32)

    def bs_step(_, carry):
        lo, hi = carry
        mid = lo + (hi - lo) // 2
        c = jnp.sum(jnp.where(u <= mid, 1.0, 0.0))
        ge = c >= float(K)
        return (jnp.where(ge, lo, mid + 1), jnp.where(ge, mid, hi))

    lo, hi = lax.fori_loop(0, 31, bs_step,
                           (jnp.int32(0), jnp.int32(2**31 - 1)))
    t = lo
    c_lt = jnp.sum(jnp.where(u < t, 1.0, 0.0))
    c_eq = jnp.sum(jnp.where(u == t, 1.0, 0.0))
    w_eq = (float(K) - c_lt) / jnp.maximum(c_eq, 1.0)
    mA = jnp.where(u < t, 1.0, jnp.where(u == t, w_eq, 0.0))
    lB = l[AR2:]
    yB = yv[AR2:]
    yA = yv[:AR2]
    mB = vb_ref[0]

    def terms(lv, yvv):
        bce = (jnp.maximum(lv, 0.0) - lv * yvv
               + jnp.log(1.0 + jnp.exp(-jnp.abs(lv))))
        pv = 1.0 / (1.0 + jnp.exp(-lv))
        return bce, pv

    bceA, pA = terms(lA, yA)
    bceB, pB = terms(lB, yB)
    s_bce = jnp.sum(mA * bceA) + jnp.sum(mB * bceB)
    s_py = jnp.sum(mA * pA * yA) + jnp.sum(mB * pB * yB)
    s_p = jnp.sum(mA * pA) + jnp.sum(mB * pB)
    s_y = jnp.sum(mA * yA) + jnp.sum(mB * yB)
    dice = 1.0 - (2.0 * s_py + 1.0) / (s_p + s_y + 1.0)
    acc[0] = acc[0] + s_bce
    acc[1] = acc[1] + dice

    @pl.when(n == N - 1)
    def _fin():
        loss_bce = acc[0] / float(N * NUM_POINTS)
        loss_dice = acc[1] / float(N)
        row = lax.broadcasted_iota(jnp.int32, (8, 128), 0)
        col = lax.broadcasted_iota(jnp.int32, (8, 128), 1)
        z = jnp.where(col == 0, loss_bce + loss_dice,
                      jnp.where(col == 1, loss_bce,
                                jnp.where(col == 2, loss_dice, 0.0)))
        out_ref[...] = jnp.where(row == 0, z, 0.0)


_tc_reduce = pl.pallas_call(
    _tc_body,
    grid=(N,),
    in_specs=[
        pl.BlockSpec((1, ROWS2, 128), lambda n: (n, 0, 0)),
        pl.BlockSpec((1, ROWS2, 128), lambda n: (n, 0, 0)),
        pl.BlockSpec((1, BR2, 128), lambda n: (n, 0, 0)),
    ],
    out_specs=pl.BlockSpec((8, 128), lambda n: (0, 0)),
    out_shape=jax.ShapeDtypeStruct((8, 128), jnp.float32),
    scratch_shapes=[pltpu.SMEM((2,), jnp.float32)],
)


def kernel(pred, target):
    pr = pred.reshape(N, H * W)
    tr = target.reshape(N, H * W)
    lv, yv = _get_sc_sample()(pr, tr, jnp.asarray(_IDX2), jnp.asarray(_WT2),
                              jnp.asarray(_BIAS2))
    out = _tc_reduce(lv.reshape(N, ROWS2, 128), yv.reshape(N, ROWS2, 128),
                     jnp.asarray(_VB3))
    return (out[0, 0], out[0, 1], out[0, 2])


# speedup vs baseline: 6.1563x; 1.7050x over previous
import functools

import numpy as np
import jax
import jax.numpy as jnp
from jax import lax
from jax.experimental import pallas as pl
from jax.experimental.pallas import tpu as pltpu
from jax.experimental.pallas import tpu_sc as plsc

N, H, W = 64, 512, 512
NUM_POINTS = 112 * 112
S = int(NUM_POINTS * 3.0)
K = int(0.75 * NUM_POINTS)
R = NUM_POINTS - K
P = S + R
NBANDS = 8
BROWS = H // NBANDS
BWORDS = (BROWS + 1) * W
f32 = np.float32


def _threefry2x32(k1, k2, x0, x1):
    u32 = np.uint32

    def rol(x, d):
        return ((x << u32(d)) | (x >> u32(32 - d))).astype(u32)

    ks = (u32(k1), u32(k2), u32(k1) ^ u32(k2) ^ u32(0x1BD11BDA))
    x0 = (x0 + ks[0]).astype(u32)
    x1 = (x1 + ks[1]).astype(u32)
    r0, r1 = (13, 15, 26, 6), (17, 29, 16, 24)
    sched = ((r0, 1, 2, 1), (r1, 2, 0, 2), (r0, 0, 1, 3),
             (r1, 1, 2, 4), (r0, 2, 0, 5))
    for rots, ia, ib, inc in sched:
        for r in rots:
            x0 = (x0 + x1).astype(u32)
            x1 = x0 ^ rol(x1, r)
        x0 = (x0 + ks[ia]).astype(u32)
        x1 = (x1 + ks[ib] + u32(inc)).astype(u32)
    return x0, x1


def _np_uniform(key, shape):
    size = int(np.prod(shape))
    io = np.arange(size, dtype=np.uint64)
    c1 = (io >> np.uint64(32)).astype(np.uint32)
    c2 = (io & np.uint64(0xFFFFFFFF)).astype(np.uint32)
    b1, b2 = _threefry2x32(key[0], key[1], c1, c2)
    bits = (b1 ^ b2).reshape(shape)
    fb = (bits >> np.uint32(9)) | np.uint32(0x3F800000)
    return (fb.view(f32) - f32(1.0)).astype(f32)


def _np_key42_split():
    key = np.array([0, 42], np.uint32)
    c1 = np.array([0, 0], np.uint32)
    c2 = np.array([0, 1], np.uint32)
    b1, b2 = _threefry2x32(key[0], key[1], c1, c2)
    return (np.array([b1[0], b2[0]], np.uint32),
            np.array([b1[1], b2[1]], np.uint32))


def _build_tables():
    k1, k2 = _np_key42_split()
    pc = _np_uniform(k1, (N, S, 2))
    rnd = _np_uniform(k2, (N, R, 2))
    coords = np.concatenate([pc, rnd], axis=1)
    x = coords[..., 0] * f32(W) - f32(0.5)
    y = coords[..., 1] * f32(H) - f32(0.5)
    x0 = np.floor(x).astype(f32)
    y0 = np.floor(y).astype(f32)
    x0c = np.clip(x0, 0, W - 1).astype(np.int64)
    y0c = np.clip(y0, 0, H - 1).astype(np.int64)
    pix = y0c * W + x0c
    ordA = np.argsort(pix[:, :S], axis=1, kind="stable")
    ordB = np.argsort(pix[:, S:], axis=1, kind="stable") + S
    order = np.concatenate([ordA, ordB], axis=1)
    tk = np.take_along_axis
    x, y, x0, y0 = (tk(a, order, axis=1) for a in (x, y, x0, y0))
    band = (np.clip(y0, 0, H - 1).astype(np.int32) // BROWS).astype(np.int32)
    wx1 = (x - x0).astype(f32); wx0 = (f32(1.0) - wx1).astype(f32)
    wy1 = (y - y0).astype(f32); wy0 = (f32(1.0) - wy1).astype(f32)
    lidx = np.empty((N, 4, P), np.int32)
    wts = np.empty((N, 4, P), f32)
    for t, (dy, dx, wy, wx) in enumerate(
            ((0, 0, wy0, wx0), (0, 1, wy0, wx1),
             (1, 0, wy1, wx0), (1, 1, wy1, wx1))):
        ix = x0 + f32(dx); iy = y0 + f32(dy)
        valid = ((ix >= 0) & (ix <= W - 1) & (iy >= 0) & (iy <= H - 1))
        ixc = np.clip(ix, 0, W - 1).astype(np.int32)
        iyc = np.clip(iy, 0, H - 1).astype(np.int32)
        lidx[:, t] = (iyc - BROWS * band) * W + ixc
        wts[:, t] = (wx * wy).astype(f32) * valid.astype(f32)
    cA = np.stack([(band[:, :S] == b).sum(axis=1) for b in range(NBANDS)])
    cB = np.stack([(band[:, S:] == b).sum(axis=1) for b in range(NBANDS)])
    capA = int(-(-int(cA.max()) // 128) * 128)
    capB = int(-(-int(cB.max()) // 128) * 128)
    cap = capA + capB
    IDX2 = np.zeros((N, NBANDS, 4 * cap), np.int32)
    WT2 = np.zeros((N, NBANDS, 4 * cap), f32)
    BIAS2 = np.zeros((N, NBANDS, cap), f32)
    BIAS2[..., :capA] = f32(1e30)
    BV = np.zeros((N, NBANDS, capB), f32)
    bnd = np.arange(NBANDS + 1)
    for n in range(N):
        eA = np.searchsorted(band[n, :S], bnd)
        eB = np.searchsorted(band[n, S:], bnd)
        i2 = IDX2[n].reshape(NBANDS, 4, cap)
        w2 = WT2[n].reshape(NBANDS, 4, cap)
        for b in range(NBANDS):
            ca = eA[b + 1] - eA[b]
            i2[b, :, :ca] = lidx[n, :, eA[b]:eA[b + 1]]
            w2[b, :, :ca] = wts[n, :, eA[b]:eA[b + 1]]
            BIAS2[n, b, :ca] = 0.0
            cb = eB[b + 1] - eB[b]
            i2[b, :, capA:capA + cb] = lidx[n, :, S + eB[b]:S + eB[b + 1]]
            w2[b, :, capA:capA + cb] = wts[n, :, S + eB[b]:S + eB[b + 1]]
            BV[n, b, :cb] = 1.0
    VB3 = BV.reshape(N, NBANDS * capB // 128, 128)
    return IDX2, WT2, BIAS2, VB3, capA, capB


_IDX2, _WT2, _BIAS2, _VB3, CAP_A, CAP_B = _build_tables()
CAP = CAP_A + CAP_B
ATOT = NBANDS * CAP_A
PTOT = NBANDS * CAP
AR2 = ATOT // 128
ROWS2 = PTOT // 128
BR2 = ROWS2 - AR2
assert (2 * BWORDS + 2 * 4 * CAP + 3 * CAP) * 4 <= 524284, "TileSpmem overflow"


@functools.cache
def _get_sc_sample():
    mesh = plsc.VectorSubcoreMesh(core_axis_name="c", subcore_axis_name="s")

    @functools.partial(
        pl.kernel,
        out_type=(jax.ShapeDtypeStruct((N, PTOT), jnp.float32),
                  jax.ShapeDtypeStruct((N, PTOT), jnp.float32)),
        mesh=mesh,
        compiler_params=pltpu.CompilerParams(needs_layout_passes=False),
        scratch_types=[
            pltpu.VMEM((BWORDS,), jnp.float32),
            pltpu.VMEM((BWORDS,), jnp.float32),
            pltpu.VMEM((4 * CAP,), jnp.int32),
            pltpu.VMEM((4 * CAP,), jnp.float32),
            pltpu.VMEM((CAP,), jnp.float32),
            pltpu.VMEM((CAP,), jnp.float32),
            pltpu.VMEM((CAP,), jnp.float32),
            pltpu.SemaphoreType.DMA,
        ],
    )
    def _sc_sample(pr, tr, idx2, wt2, bias2, lout, yout,
                   bandp, bandt, idxb, wtb, biasb, lbuf, ybuf, sem):
        wid = lax.axis_index("s") * 2 + lax.axis_index("c")
        for ii in range(2):
            img = wid * 2 + ii
            for b in range(NBANDS):
                rows = BROWS + 1 if b < NBANDS - 1 else BROWS
                nw = rows * W
                cps = [
                    pltpu.async_copy(pr.at[img, pl.ds(b * BROWS * W, nw)],
                                     bandp.at[pl.ds(0, nw)], sem),
                    pltpu.async_copy(tr.at[img, pl.ds(b * BROWS * W, nw)],
                                     bandt.at[pl.ds(0, nw)], sem),
                    pltpu.async_copy(idx2.at[img, b], idxb, sem),
                    pltpu.async_copy(wt2.at[img, b], wtb, sem),
                    pltpu.async_copy(bias2.at[img, b], biasb, sem),
                ]
                for cp in cps:
                    cp.wait()

                def group(g, carry):
                    o = g * 16
                    accl = biasb[pl.ds(o, 16)]
                    accy = jnp.zeros((16,), jnp.float32)
                    for t in range(4):
                        iv = idxb[pl.ds(t * CAP + o, 16)]
                        wv = wtb[pl.ds(t * CAP + o, 16)]
                        pv = plsc.load_gather(bandp, [iv])
                        tv = plsc.load_gather(bandt, [iv])
                        accl = accl + wv * pv
                        accy = accy + wv * tv
                    lbuf[pl.ds(o, 16)] = accl
                    ybuf[pl.ds(o, 16)] = accy
                    return carry

                lax.fori_loop(0, CAP // 16, group, 0)
                pltpu.sync_copy(lbuf.at[pl.ds(0, CAP_A)],
                                lout.at[img, pl.ds(b * CAP_A, CAP_A)])
                pltpu.sync_copy(ybuf.at[pl.ds(0, CAP_A)],
                                yout.at[img, pl.ds(b * CAP_A, CAP_A)])
                pltpu.sync_copy(lbuf.at[pl.ds(CAP_A, CAP_B)],
                                lout.at[img, pl.ds(ATOT + b * CAP_B, CAP_B)])
                pltpu.sync_copy(ybuf.at[pl.ds(CAP_A, CAP_B)],
                                yout.at[img, pl.ds(ATOT + b * CAP_B, CAP_B)])

    return _sc_sample


TCG = 16
TCSTEPS = N // TCG


def _tc_body(l_ref, y_ref, vb_ref, out_ref, acc):
    step = pl.program_id(0)

    @pl.when(step == 0)
    def _init():
        acc[0] = 0.0
        acc[1] = 0.0

    l = l_ref[...]
    yv = y_ref[...]
    lA = l[:, :AR2]
    u = lax.bitcast_convert_type(jnp.abs(lA), jnp.int32)

    def cnt(pred_mask):
        return jnp.sum(jnp.where(pred_mask, 1.0, 0.0), axis=(1, 2),
                       keepdims=True)

    def bs_step(_, carry):
        lo, hi = carry
        mid = lo + (hi - lo) // 2
        ge = cnt(u <= mid) >= float(K)
        return (jnp.where(ge, lo, mid + 1), jnp.where(ge, mid, hi))

    init = (jnp.zeros((TCG, 1, 1), jnp.int32),
            jnp.full((TCG, 1, 1), 2**31 - 1, jnp.int32))
    lo, hi = lax.fori_loop(0, 31, bs_step, init)
    t = lo
    c_lt = cnt(u < t)
    c_eq = cnt(u == t)
    w_eq = (float(K) - c_lt) / jnp.maximum(c_eq, 1.0)
    mA = jnp.where(u < t, 1.0, jnp.where(u == t, w_eq, 0.0))
    lB = l[:, AR2:]
    yB = yv[:, AR2:]
    yA = yv[:, :AR2]
    mB = vb_ref[...]

    def terms(lv, yvv):
        bce = (jnp.maximum(lv, 0.0) - lv * yvv
               + jnp.log(1.0 + jnp.exp(-jnp.abs(lv))))
        pv = 1.0 / (1.0 + jnp.exp(-lv))
        return bce, pv

    def psum(x):
        return jnp.sum(x, axis=(1, 2), keepdims=True)

    bceA, pA = terms(lA, yA)
    bceB, pB = terms(lB, yB)
    s_bce = jnp.sum(mA * bceA) + jnp.sum(mB * bceB)
    s_py = psum(mA * pA * yA) + psum(mB * pB * yB)
    s_p = psum(mA * pA) + psum(mB * pB)
    s_y = psum(mA * yA) + psum(mB * yB)
    dice = 1.0 - (2.0 * s_py + 1.0) / (s_p + s_y + 1.0)
    acc[0] = acc[0] + s_bce
    acc[1] = acc[1] + jnp.sum(dice)

    @pl.when(step == TCSTEPS - 1)
    def _fin():
        loss_bce = acc[0] / float(N * NUM_POINTS)
        loss_dice = acc[1] / float(N)
        row = lax.broadcasted_iota(jnp.int32, (8, 128), 0)
        col = lax.broadcasted_iota(jnp.int32, (8, 128), 1)
        z = jnp.where(col == 0, loss_bce + loss_dice,
                      jnp.where(col == 1, loss_bce,
                                jnp.where(col == 2, loss_dice, 0.0)))
        out_ref[...] = jnp.where(row == 0, z, 0.0)


_tc_reduce = pl.pallas_call(
    _tc_body,
    grid=(TCSTEPS,),
    in_specs=[
        pl.BlockSpec((TCG, ROWS2, 128), lambda i: (i, 0, 0)),
        pl.BlockSpec((TCG, ROWS2, 128), lambda i: (i, 0, 0)),
        pl.BlockSpec((TCG, BR2, 128), lambda i: (i, 0, 0)),
    ],
    out_specs=pl.BlockSpec((8, 128), lambda i: (0, 0)),
    out_shape=jax.ShapeDtypeStruct((8, 128), jnp.float32),
    scratch_shapes=[pltpu.SMEM((2,), jnp.float32)],
)


def kernel(pred, target):
    pr = pred.reshape(N, H * W)
    tr = target.reshape(N, H * W)
    lv, yv = _get_sc_sample()(pr, tr, jnp.asarray(_IDX2), jnp.asarray(_WT2),
                              jnp.asarray(_BIAS2))
    out = _tc_reduce(lv.reshape(N, ROWS2, 128), yv.reshape(N, ROWS2, 128),
                     jnp.asarray(_VB3))
    return (out[0, 0], out[0, 1], out[0, 2])


# speedup vs baseline: 6.8675x; 1.1155x over previous
import functools

import numpy as np
import jax
import jax.numpy as jnp
from jax import lax
from jax.experimental import pallas as pl
from jax.experimental.pallas import tpu as pltpu
from jax.experimental.pallas import tpu_sc as plsc

N, H, W = 64, 512, 512
NUM_POINTS = 112 * 112
S = int(NUM_POINTS * 3.0)
K = int(0.75 * NUM_POINTS)
R = NUM_POINTS - K
P = S + R
NBANDS = 8
BROWS = H // NBANDS
BWORDS = (BROWS + 1) * W
f32 = np.float32


def _threefry2x32(k1, k2, x0, x1):
    u32 = np.uint32

    def rol(x, d):
        return ((x << u32(d)) | (x >> u32(32 - d))).astype(u32)

    ks = (u32(k1), u32(k2), u32(k1) ^ u32(k2) ^ u32(0x1BD11BDA))
    x0 = (x0 + ks[0]).astype(u32)
    x1 = (x1 + ks[1]).astype(u32)
    r0, r1 = (13, 15, 26, 6), (17, 29, 16, 24)
    sched = ((r0, 1, 2, 1), (r1, 2, 0, 2), (r0, 0, 1, 3),
             (r1, 1, 2, 4), (r0, 2, 0, 5))
    for rots, ia, ib, inc in sched:
        for r in rots:
            x0 = (x0 + x1).astype(u32)
            x1 = x0 ^ rol(x1, r)
        x0 = (x0 + ks[ia]).astype(u32)
        x1 = (x1 + ks[ib] + u32(inc)).astype(u32)
    return x0, x1


def _np_uniform(key, shape):
    size = int(np.prod(shape))
    io = np.arange(size, dtype=np.uint64)
    c1 = (io >> np.uint64(32)).astype(np.uint32)
    c2 = (io & np.uint64(0xFFFFFFFF)).astype(np.uint32)
    b1, b2 = _threefry2x32(key[0], key[1], c1, c2)
    bits = (b1 ^ b2).reshape(shape)
    fb = (bits >> np.uint32(9)) | np.uint32(0x3F800000)
    return (fb.view(f32) - f32(1.0)).astype(f32)


def _np_key42_split():
    key = np.array([0, 42], np.uint32)
    c1 = np.array([0, 0], np.uint32)
    c2 = np.array([0, 1], np.uint32)
    b1, b2 = _threefry2x32(key[0], key[1], c1, c2)
    return (np.array([b1[0], b2[0]], np.uint32),
            np.array([b1[1], b2[1]], np.uint32))


def _build_tables():
    k1, k2 = _np_key42_split()
    pc = _np_uniform(k1, (N, S, 2))
    rnd = _np_uniform(k2, (N, R, 2))
    coords = np.concatenate([pc, rnd], axis=1)
    x = coords[..., 0] * f32(W) - f32(0.5)
    y = coords[..., 1] * f32(H) - f32(0.5)
    x0 = np.floor(x).astype(f32)
    y0 = np.floor(y).astype(f32)
    x0c = np.clip(x0, 0, W - 1).astype(np.int64)
    y0c = np.clip(y0, 0, H - 1).astype(np.int64)
    pix = y0c * W + x0c
    ordA = np.argsort(pix[:, :S], axis=1, kind="stable")
    ordB = np.argsort(pix[:, S:], axis=1, kind="stable") + S
    order = np.concatenate([ordA, ordB], axis=1)
    tk = np.take_along_axis
    x, y, x0, y0 = (tk(a, order, axis=1) for a in (x, y, x0, y0))
    band = (np.clip(y0, 0, H - 1).astype(np.int32) // BROWS).astype(np.int32)
    wx1 = (x - x0).astype(f32); wx0 = (f32(1.0) - wx1).astype(f32)
    wy1 = (y - y0).astype(f32); wy0 = (f32(1.0) - wy1).astype(f32)
    lidx = np.empty((N, 4, P), np.int32)
    wts = np.empty((N, 4, P), f32)
    for t, (dy, dx, wy, wx) in enumerate(
            ((0, 0, wy0, wx0), (0, 1, wy0, wx1),
             (1, 0, wy1, wx0), (1, 1, wy1, wx1))):
        ix = x0 + f32(dx); iy = y0 + f32(dy)
        valid = ((ix >= 0) & (ix <= W - 1) & (iy >= 0) & (iy <= H - 1))
        ixc = np.clip(ix, 0, W - 1).astype(np.int32)
        iyc = np.clip(iy, 0, H - 1).astype(np.int32)
        lidx[:, t] = (iyc - BROWS * band) * W + ixc
        wts[:, t] = (wx * wy).astype(f32) * valid.astype(f32)
    cA = np.stack([(band[:, :S] == b).sum(axis=1) for b in range(NBANDS)])
    cB = np.stack([(band[:, S:] == b).sum(axis=1) for b in range(NBANDS)])
    capA = int(-(-int(cA.max()) // 128) * 128)
    capB = int(-(-int(cB.max()) // 128) * 128)
    cap = capA + capB
    IDX2 = np.zeros((N, NBANDS, 4 * cap), np.int32)
    WT2 = np.zeros((N, NBANDS, 4 * cap), f32)
    BIAS2 = np.zeros((N, NBANDS, cap), f32)
    BIAS2[..., :capA] = f32(1e30)
    BV = np.zeros((N, NBANDS, capB), f32)
    bnd = np.arange(NBANDS + 1)
    for n in range(N):
        eA = np.searchsorted(band[n, :S], bnd)
        eB = np.searchsorted(band[n, S:], bnd)
        i2 = IDX2[n].reshape(NBANDS, 4, cap)
        w2 = WT2[n].reshape(NBANDS, 4, cap)
        for b in range(NBANDS):
            ca = eA[b + 1] - eA[b]
            i2[b, :, :ca] = lidx[n, :, eA[b]:eA[b + 1]]
            w2[b, :, :ca] = wts[n, :, eA[b]:eA[b + 1]]
            BIAS2[n, b, :ca] = 0.0
            cb = eB[b + 1] - eB[b]
            i2[b, :, capA:capA + cb] = lidx[n, :, S + eB[b]:S + eB[b + 1]]
            w2[b, :, capA:capA + cb] = wts[n, :, S + eB[b]:S + eB[b + 1]]
            BV[n, b, :cb] = 1.0
    VB3 = BV.reshape(N, NBANDS * capB // 128, 128)
    return IDX2, WT2, BIAS2, VB3, capA, capB


_IDX2, _WT2, _BIAS2, _VB3, CAP_A, CAP_B = _build_tables()
CAP = CAP_A + CAP_B
ATOT = NBANDS * CAP_A
PTOT = NBANDS * CAP
AR2 = ATOT // 128
ROWS2 = PTOT // 128
BR2 = ROWS2 - AR2
assert (2 * BWORDS + 2 * 4 * CAP + 3 * CAP) * 4 <= 524284, "TileSpmem overflow"


@functools.cache
def _get_sc_sample():
    mesh = plsc.VectorSubcoreMesh(core_axis_name="c", subcore_axis_name="s")

    @functools.partial(
        pl.kernel,
        out_type=(jax.ShapeDtypeStruct((N, PTOT), jnp.float32),
                  jax.ShapeDtypeStruct((N, PTOT), jnp.float32)),
        mesh=mesh,
        compiler_params=pltpu.CompilerParams(needs_layout_passes=False),
        scratch_types=[
            pltpu.VMEM((BWORDS,), jnp.float32),
            pltpu.VMEM((BWORDS,), jnp.float32),
            pltpu.VMEM((4 * CAP,), jnp.int32),
            pltpu.VMEM((4 * CAP,), jnp.float32),
            pltpu.VMEM((CAP,), jnp.float32),
            pltpu.VMEM((CAP,), jnp.float32),
            pltpu.VMEM((CAP,), jnp.float32),
            pltpu.SemaphoreType.DMA,
        ],
    )
    def _sc_sample(pr, tr, idx2, wt2, bias2, lout, yout,
                   bandp, bandt, idxb, wtb, biasb, lbuf, ybuf, sem):
        wid = lax.axis_index("s") * 2 + lax.axis_index("c")
        for ii in range(2):
            img = wid * 2 + ii
            for b in range(NBANDS):
                rows = BROWS + 1 if b < NBANDS - 1 else BROWS
                nw = rows * W
                cps = [
                    pltpu.async_copy(pr.at[img, pl.ds(b * BROWS * W, nw)],
                                     bandp.at[pl.ds(0, nw)], sem),
                    pltpu.async_copy(tr.at[img, pl.ds(b * BROWS * W, nw)],
                                     bandt.at[pl.ds(0, nw)], sem),
                    pltpu.async_copy(idx2.at[img, b], idxb, sem),
                    pltpu.async_copy(wt2.at[img, b], wtb, sem),
                    pltpu.async_copy(bias2.at[img, b], biasb, sem),
                ]
                for cp in cps:
                    cp.wait()

                @plsc.parallel_loop(0, CAP, 16, unroll=2)
                def group(o):
                    accl = biasb[pl.ds(o, 16)]
                    accy = jnp.zeros((16,), jnp.float32)
                    for t in range(4):
                        iv = idxb[pl.ds(t * CAP + o, 16)]
                        wv = wtb[pl.ds(t * CAP + o, 16)]
                        pv = plsc.load_gather(bandp, [iv])
                        tv = plsc.load_gather(bandt, [iv])
                        accl = accl + wv * pv
                        accy = accy + wv * tv
                    lbuf[pl.ds(o, 16)] = accl
                    ybuf[pl.ds(o, 16)] = accy
                pltpu.sync_copy(lbuf.at[pl.ds(0, CAP_A)],
                                lout.at[img, pl.ds(b * CAP_A, CAP_A)])
                pltpu.sync_copy(ybuf.at[pl.ds(0, CAP_A)],
                                yout.at[img, pl.ds(b * CAP_A, CAP_A)])
                pltpu.sync_copy(lbuf.at[pl.ds(CAP_A, CAP_B)],
                                lout.at[img, pl.ds(ATOT + b * CAP_B, CAP_B)])
                pltpu.sync_copy(ybuf.at[pl.ds(CAP_A, CAP_B)],
                                yout.at[img, pl.ds(ATOT + b * CAP_B, CAP_B)])

    return _sc_sample


TCG = 16
TCSTEPS = N // TCG


def _tc_body(l_ref, y_ref, vb_ref, out_ref, acc):
    step = pl.program_id(0)

    @pl.when(step == 0)
    def _init():
        acc[0] = 0.0
        acc[1] = 0.0

    l = l_ref[...]
    yv = y_ref[...]
    lA = l[:, :AR2]
    u = lax.shift_right_logical(
        lax.bitcast_convert_type(jnp.abs(lA), jnp.int32), 11)

    def cnt(pred_mask):
        return jnp.sum(jnp.where(pred_mask, 1.0, 0.0), axis=(1, 2),
                       keepdims=True)

    def bs_step(_, carry):
        lo, hi = carry
        mid = lo + (hi - lo) // 2
        ge = cnt(u <= mid) >= float(K)
        return (jnp.where(ge, lo, mid + 1), jnp.where(ge, mid, hi))

    init = (jnp.zeros((TCG, 1, 1), jnp.int32),
            jnp.full((TCG, 1, 1), 2**20 - 1, jnp.int32))
    lo, hi = lax.fori_loop(0, 20, bs_step, init)
    t = lo
    c_lt = cnt(u < t)
    c_eq = cnt(u == t)
    w_eq = (float(K) - c_lt) / jnp.maximum(c_eq, 1.0)
    mA = jnp.where(u < t, 1.0, jnp.where(u == t, w_eq, 0.0))
    lB = l[:, AR2:]
    yB = yv[:, AR2:]
    yA = yv[:, :AR2]
    mB = vb_ref[...]

    def terms(lv, yvv):
        bce = (jnp.maximum(lv, 0.0) - lv * yvv
               + jnp.log(1.0 + jnp.exp(-jnp.abs(lv))))
        pv = 1.0 / (1.0 + jnp.exp(-lv))
        return bce, pv

    def psum(x):
        return jnp.sum(x, axis=(1, 2), keepdims=True)

    bceA, pA = terms(lA, yA)
    bceB, pB = terms(lB, yB)
    s_bce = jnp.sum(mA * bceA) + jnp.sum(mB * bceB)
    s_py = psum(mA * pA * yA) + psum(mB * pB * yB)
    s_p = psum(mA * pA) + psum(mB * pB)
    s_y = psum(mA * yA) + psum(mB * yB)
    dice = 1.0 - (2.0 * s_py + 1.0) / (s_p + s_y + 1.0)
    acc[0] = acc[0] + s_bce
    acc[1] = acc[1] + jnp.sum(dice)

    @pl.when(step == TCSTEPS - 1)
    def _fin():
        loss_bce = acc[0] / float(N * NUM_POINTS)
        loss_dice = acc[1] / float(N)
        row = lax.broadcasted_iota(jnp.int32, (8, 128), 0)
        col = lax.broadcasted_iota(jnp.int32, (8, 128), 1)
        z = jnp.where(col == 0, loss_bce + loss_dice,
                      jnp.where(col == 1, loss_bce,
                                jnp.where(col == 2, loss_dice, 0.0)))
        out_ref[...] = jnp.where(row == 0, z, 0.0)


_tc_reduce = pl.pallas_call(
    _tc_body,
    grid=(TCSTEPS,),
    in_specs=[
        pl.BlockSpec((TCG, ROWS2, 128), lambda i: (i, 0, 0)),
        pl.BlockSpec((TCG, ROWS2, 128), lambda i: (i, 0, 0)),
        pl.BlockSpec((TCG, BR2, 128), lambda i: (i, 0, 0)),
    ],
    out_specs=pl.BlockSpec((8, 128), lambda i: (0, 0)),
    out_shape=jax.ShapeDtypeStruct((8, 128), jnp.float32),
    scratch_shapes=[pltpu.SMEM((2,), jnp.float32)],
)


def kernel(pred, target):
    pr = pred.reshape(N, H * W)
    tr = target.reshape(N, H * W)
    lv, yv = _get_sc_sample()(pr, tr, jnp.asarray(_IDX2), jnp.asarray(_WT2),
                              jnp.asarray(_BIAS2))
    out = _tc_reduce(lv.reshape(N, ROWS2, 128), yv.reshape(N, ROWS2, 128),
                     jnp.asarray(_VB3))
    return (out[0, 0], out[0, 1], out[0, 2])
